# baseline (speedup 1.0000x reference)
import jax
import jax.numpy as jnp
from jax import lax
from jax.experimental import pallas as pl
from jax.experimental.pallas import tpu as pltpu

CB = 8
NBUF = 8


def _body(ctx1_ref, ctx2_ref, w11_ref, b11_ref, w12_ref, b12_ref,
          w21_ref, b21_ref, w22_ref, b22_ref, x_hbm, o1_hbm, o2_hbm,
          g1_scr, g2_scr, xbuf, o1buf, o2buf, in_sem, o1_sem, o2_sem):
    B = x_hbm.shape[0]
    nc = B // CB

    def in_copy(c, slot):
        return pltpu.make_async_copy(
            x_hbm.at[pl.ds(c * CB, CB)], xbuf.at[slot], in_sem.at[slot])

    def o1_copy(c, slot):
        return pltpu.make_async_copy(
            o1buf.at[slot], o1_hbm.at[pl.ds(c * CB, CB)], o1_sem.at[slot])

    def o2_copy(c, slot):
        return pltpu.make_async_copy(
            o2buf.at[slot], o2_hbm.at[pl.ds(c * CB, CB)], o2_sem.at[slot])

    for s in range(NBUF):
        in_copy(s, s).start()

    h1 = jnp.maximum(
        jnp.dot(ctx1_ref[...], w11_ref[...],
                preferred_element_type=jnp.float32) + b11_ref[...], 0.0)
    g1_scr[...] = jax.nn.sigmoid(
        jnp.dot(h1, w12_ref[...],
                preferred_element_type=jnp.float32) + b12_ref[...]) * 2.0
    h2 = jnp.maximum(
        jnp.dot(ctx2_ref[...], w21_ref[...],
                preferred_element_type=jnp.float32) + b21_ref[...], 0.0)
    g2_scr[...] = jax.nn.sigmoid(
        jnp.dot(h2, w22_ref[...],
                preferred_element_type=jnp.float32) + b22_ref[...]) * 2.0

    def step(c, carry):
        slot = lax.rem(c, NBUF)
        in_copy(c, slot).wait()

        @pl.when(c >= NBUF)
        def _():
            o1_copy(c - NBUF, slot).wait()
            o2_copy(c - NBUF, slot).wait()

        x = xbuf[slot]
        g1 = g1_scr[...][None]
        g2 = g2_scr[...][None]
        o1buf[slot] = x * g1
        o2buf[slot] = x * g2
        o1_copy(c, slot).start()
        o2_copy(c, slot).start()

        @pl.when(c + NBUF < nc)
        def _():
            in_copy(c + NBUF, slot).start()

        return carry

    lax.fori_loop(0, nc, step, 0)

    for s in range(NBUF):
        c = nc - NBUF + s
        o1_copy(c, c % NBUF).wait()
        o2_copy(c, c % NBUF).wait()


def kernel(feed_dict, flat_emb, fs1_ctx_bias, fs2_ctx_bias,
           fs1_W1, fs1_b1, fs1_W2, fs1_b2,
           fs2_W1, fs2_b1, fs2_W2, fs2_b2):
    B, L, D = flat_emb.shape
    E = fs1_ctx_bias.shape[-1]
    H = fs1_W1.shape[-1]

    vmem = pl.BlockSpec(memory_space=pltpu.MemorySpace.VMEM)
    hbm = pl.BlockSpec(memory_space=pltpu.MemorySpace.HBM)

    out1, out2 = pl.pallas_call(
        _body,
        in_specs=[vmem] * 10 + [hbm],
        out_specs=[hbm, hbm],
        out_shape=[
            jax.ShapeDtypeStruct((B, L, D), jnp.float32),
            jax.ShapeDtypeStruct((B, L, D), jnp.float32),
        ],
        scratch_shapes=[
            pltpu.VMEM((1, D), jnp.float32),
            pltpu.VMEM((1, D), jnp.float32),
            pltpu.VMEM((NBUF, CB, L, D), jnp.float32),
            pltpu.VMEM((NBUF, CB, L, D), jnp.float32),
            pltpu.VMEM((NBUF, CB, L, D), jnp.float32),
            pltpu.SemaphoreType.DMA((NBUF,)),
            pltpu.SemaphoreType.DMA((NBUF,)),
            pltpu.SemaphoreType.DMA((NBUF,)),
        ],
    )(fs1_ctx_bias, fs2_ctx_bias,
      fs1_W1, fs1_b1.reshape(1, H), fs1_W2, fs1_b2.reshape(1, D),
      fs2_W1, fs2_b1.reshape(1, H), fs2_W2, fs2_b2.reshape(1, D),
      flat_emb)

    return (out1, out2)


# speedup vs baseline: 1.0077x; 1.0077x over previous
import jax
import jax.numpy as jnp
from jax import lax
from jax.experimental import pallas as pl
from jax.experimental.pallas import tpu as pltpu

CB = 8
NBUF = 8


def _body(ctx1_ref, ctx2_ref, w11_ref, b11_ref, w12_ref, b12_ref,
          w21_ref, b21_ref, w22_ref, b22_ref, x_hbm, o1_hbm, o2_hbm,
          g1_scr, g2_scr, xbuf, o1buf, o2buf, in_sem, o1_sem, o2_sem):
    B = x_hbm.shape[0]
    nc = B // CB

    def in_copy(c, slot):
        return pltpu.make_async_copy(
            x_hbm.at[pl.ds(c * CB, CB)], xbuf.at[slot], in_sem.at[slot])

    def o1_copy(c, slot):
        return pltpu.make_async_copy(
            o1buf.at[slot], o1_hbm.at[pl.ds(c * CB, CB)], o1_sem.at[slot])

    def o2_copy(c, slot):
        return pltpu.make_async_copy(
            o2buf.at[slot], o2_hbm.at[pl.ds(c * CB, CB)], o2_sem.at[slot])

    for s in range(NBUF):
        in_copy(s, s).start(priority=s % 2)

    h1 = jnp.maximum(
        jnp.dot(ctx1_ref[...], w11_ref[...],
                preferred_element_type=jnp.float32) + b11_ref[...], 0.0)
    g1_scr[...] = jax.nn.sigmoid(
        jnp.dot(h1, w12_ref[...],
                preferred_element_type=jnp.float32) + b12_ref[...]) * 2.0
    h2 = jnp.maximum(
        jnp.dot(ctx2_ref[...], w21_ref[...],
                preferred_element_type=jnp.float32) + b21_ref[...], 0.0)
    g2_scr[...] = jax.nn.sigmoid(
        jnp.dot(h2, w22_ref[...],
                preferred_element_type=jnp.float32) + b22_ref[...]) * 2.0

    def round_body(r, carry):
        for s in range(NBUF):
            c = r * NBUF + s
            in_copy(c, s).wait()

            @pl.when(r >= 1)
            def _():
                o1_copy(c - NBUF, s).wait()
                o2_copy(c - NBUF, s).wait()

            x = xbuf[s]
            g1 = g1_scr[...][None]
            g2 = g2_scr[...][None]
            o1buf[s] = x * g1
            o2buf[s] = x * g2
            o1_copy(c, s).start(priority=0)
            o2_copy(c, s).start(priority=1)

            @pl.when(c + NBUF < nc)
            def _():
                in_copy(c + NBUF, s).start(priority=s % 2)

        return carry

    lax.fori_loop(0, nc // NBUF, round_body, 0)

    for s in range(NBUF):
        c = nc - NBUF + s
        o1_copy(c, c % NBUF).wait()
        o2_copy(c, c % NBUF).wait()


def kernel(feed_dict, flat_emb, fs1_ctx_bias, fs2_ctx_bias,
           fs1_W1, fs1_b1, fs1_W2, fs1_b2,
           fs2_W1, fs2_b1, fs2_W2, fs2_b2):
    B, L, D = flat_emb.shape
    E = fs1_ctx_bias.shape[-1]
    H = fs1_W1.shape[-1]

    vmem = pl.BlockSpec(memory_space=pltpu.MemorySpace.VMEM)
    hbm = pl.BlockSpec(memory_space=pltpu.MemorySpace.HBM)

    out1, out2 = pl.pallas_call(
        _body,
        in_specs=[vmem] * 10 + [hbm],
        out_specs=[hbm, hbm],
        out_shape=[
            jax.ShapeDtypeStruct((B, L, D), jnp.float32),
            jax.ShapeDtypeStruct((B, L, D), jnp.float32),
        ],
        scratch_shapes=[
            pltpu.VMEM((1, D), jnp.float32),
            pltpu.VMEM((1, D), jnp.float32),
            pltpu.VMEM((NBUF, CB, L, D), jnp.float32),
            pltpu.VMEM((NBUF, CB, L, D), jnp.float32),
            pltpu.VMEM((NBUF, CB, L, D), jnp.float32),
            pltpu.SemaphoreType.DMA((NBUF,)),
            pltpu.SemaphoreType.DMA((NBUF,)),
            pltpu.SemaphoreType.DMA((NBUF,)),
        ],
    )(fs1_ctx_bias, fs2_ctx_bias,
      fs1_W1, fs1_b1.reshape(1, H), fs1_W2, fs1_b2.reshape(1, D),
      fs2_W1, fs2_b1.reshape(1, H), fs2_W2, fs2_b2.reshape(1, D),
      flat_emb)

    return (out1, out2)
